# Initial kernel scaffold; baseline (speedup 1.0000x reference)
import jax
import jax.numpy as jnp
from jax.experimental import pallas as pl


def kernel(index, outputs, label, out, u, prevSimilarity, masterVector, cat_labels, flag, epoch):
    raise NotImplementedError("write your pallas kernel here")


# speedup vs baseline: 51.4802x; 51.4802x over previous
import dataclasses
import functools

import jax
import jax.numpy as jnp
import numpy as np
from jax import lax
from jax.experimental import pallas as pl
from jax.experimental.pallas import tpu as pltpu
from jax.experimental.pallas import tpu_sc as plsc

NUM_EXAMP = 50000
NUM_CLASSES = 100
BATCH = 1024
FEAT = 512
EPS = 1e-4

NSUB = 16
N_PAD = 50176
CHUNK = N_PAD // NSUB
NVREG = CHUNK // 16
C_PAD = 112
NBIN = 16
HROW = NBIN * C_PAD
MININT = np.int32(-2147483648)


def _zero_table(ref, nwords):
    zeros = jnp.zeros((16,), jnp.int32)

    @pl.loop(0, nwords // 16)
    def _(q):
        ref[pl.ds(q * 16, 16)] = zeros


def _reduce_hist(all_h, dst):
    @pl.loop(0, HROW // 16)
    def _(q):
        acc = all_h[pl.ds(q * 16, 16)]
        for s in range(1, NSUB):
            acc += all_h[pl.ds(s * HROW + q * 16, 16)]
        dst[pl.ds(q * 16, 16)] = acc


def _radix_update(cnt, p_ref, r_ref):
    for cg in range(C_PAD // 16):
        r_v = r_ref[pl.ds(cg * 16, 16)]
        p_v = p_ref[pl.ds(cg * 16, 16)]
        cum = jnp.zeros((16,), jnp.int32)
        bsel = jnp.zeros((16,), jnp.int32)
        newr = r_v
        done = jnp.zeros((16,), jnp.bool_)
        for b in range(NBIN):
            tot = cnt[pl.ds(b * C_PAD + cg * 16, 16)]
            prev_cum = cum
            cum = cum + tot
            take = jnp.logical_and(jnp.logical_not(done), r_v < cum)
            bsel = jnp.where(take, jnp.int32(b), bsel)
            newr = jnp.where(take, r_v - prev_cum, newr)
            done = jnp.logical_or(done, take)
        p_ref[pl.ds(cg * 16, 16)] = p_v * NBIN + bsel
        r_ref[pl.ds(cg * 16, 16)] = newr


def _sc_select(u_pad, cat_pad, index, perc16):
    mesh = plsc.VectorSubcoreMesh(core_axis_name="c", subcore_axis_name="s")
    cp = pltpu.CompilerParams()
    if "needs_layout_passes" in pltpu.CompilerParams.__dataclass_fields__:
        cp = dataclasses.replace(cp, needs_layout_passes=False)

    @functools.partial(
        pl.kernel,
        mesh=mesh,
        compiler_params=cp,
        out_type=[
            jax.ShapeDtypeStruct((N_PAD,), jnp.float32),
            jax.ShapeDtypeStruct((C_PAD,), jnp.float32),
            jax.ShapeDtypeStruct((BATCH,), jnp.float32),
        ],
        scratch_types=[
            pltpu.VMEM((CHUNK,), jnp.float32),
            pltpu.VMEM((CHUNK,), jnp.int32),
            pltpu.VMEM((CHUNK,), jnp.int32),
            pltpu.VMEM((CHUNK,), jnp.float32),
            pltpu.VMEM((C_PAD,), jnp.int32),
            pltpu.VMEM((C_PAD,), jnp.int32),
            pltpu.VMEM((C_PAD,), jnp.int32),
            pltpu.VMEM((C_PAD,), jnp.int32),
            pltpu.VMEM((C_PAD,), jnp.int32),
            pltpu.VMEM((C_PAD,), jnp.float32),
            pltpu.VMEM((HROW,), jnp.int32),
            pltpu.VMEM((NSUB * HROW,), jnp.int32),
            pltpu.VMEM((16,), jnp.float32),
            pltpu.VMEM((N_PAD,), jnp.float32),
            pltpu.VMEM((BATCH // NSUB,), jnp.int32),
            pltpu.VMEM((BATCH // NSUB,), jnp.float32),
            pltpu.VMEM_SHARED((NSUB * HROW,), jnp.int32),
        ],
    )
    def sel(u_hbm, cat_hbm, idx_hbm, perc_hbm, w_hbm, kf_hbm, ub_hbm,
            utmp, keyb, catb, wb, p_b, p2_b, r_b, n_b, as_b, kf_b,
            cntb, all_h, perc, ufull, ivb, ubb, hsh):
        cid = lax.axis_index("c")
        sid = lax.axis_index("s")
        ones_i = jnp.ones((16,), jnp.int32)

        @pl.when(cid == 0)
        def _core0():
            base = sid * CHUNK
            pltpu.sync_copy(u_hbm.at[pl.ds(base, CHUNK)], utmp)
            pltpu.sync_copy(cat_hbm.at[pl.ds(base, CHUNK)], catb)
            pltpu.sync_copy(perc_hbm, perc)

            @pl.loop(0, NVREG)
            def _(i):
                fb = plsc.bitcast(utmp[pl.ds(i * 16, 16)], jnp.int32)
                m = lax.shift_right_arithmetic(fb, 31)
                keyb[pl.ds(i * 16, 16)] = lax.bitwise_xor(
                    fb, lax.bitwise_or(m, MININT))

            _zero_table(cntb, HROW)

            @pl.loop(0, NVREG)
            def _(i):
                cv = catb[pl.ds(i * 16, 16)]
                plsc.addupdate_scatter(cntb, [cv], ones_i)

            pltpu.sync_copy(cntb, hsh.at[pl.ds(sid * HROW, HROW)])
            plsc.subcore_barrier()
            pltpu.sync_copy(hsh, all_h)
            plsc.subcore_barrier()
            _reduce_hist(all_h, cntb)
            pv16 = perc[pl.ds(0, 16)]
            for cg in range(C_PAD // 16):
                n_v = cntb[pl.ds(cg * 16, 16)]
                n_b[pl.ds(cg * 16, 16)] = n_v
                nf = n_v.astype(jnp.float32)
                kf = (nf / jnp.float32(100.0)) * pv16
                ki = kf.astype(jnp.int32)
                kf_b[pl.ds(cg * 16, 16)] = ki.astype(jnp.float32)
                r_b[pl.ds(cg * 16, 16)] = ki
                p_b[pl.ds(cg * 16, 16)] = jnp.zeros((16,), jnp.int32)
                as_b[pl.ds(cg * 16, 16)] = jnp.where(
                    ki >= n_v, ones_i, jnp.zeros((16,), jnp.int32))

            for rnd in range(8):
                shift = 32 - 4 * (rnd + 1)
                _zero_table(cntb, HROW)

                @pl.loop(0, NVREG)
                def _(i, _shift=shift):
                    kv = keyb[pl.ds(i * 16, 16)]
                    cv = catb[pl.ds(i * 16, 16)]
                    val = lax.shift_right_logical(kv, _shift)
                    pv = plsc.load_gather(p_b, [cv])
                    match = lax.shift_right_logical(val, 4) == pv
                    fidx = lax.bitwise_and(val, NBIN - 1) * C_PAD + cv
                    plsc.addupdate_scatter(cntb, [fidx], ones_i, mask=match)

                pltpu.sync_copy(cntb, hsh.at[pl.ds(sid * HROW, HROW)])
                plsc.subcore_barrier()
                pltpu.sync_copy(hsh, all_h)
                plsc.subcore_barrier()
                _reduce_hist(all_h, cntb)
                _radix_update(cntb, p_b, r_b)

            for cg in range(C_PAD // 16):
                p2_b[pl.ds(cg * 16, 16)] = jnp.zeros((16,), jnp.int32)
            for rnd in range(4):
                shift = 16 - 4 * (rnd + 1)
                _zero_table(cntb, HROW)

                @pl.loop(0, NVREG)
                def _(i, _shift=shift):
                    kv = keyb[pl.ds(i * 16, 16)]
                    cv = catb[pl.ds(i * 16, 16)]
                    tv = plsc.load_gather(p_b, [cv])
                    jv = base + i * 16 + lax.broadcasted_iota(jnp.int32, (16,), 0)
                    val = lax.shift_right_logical(jv, _shift)
                    p2v = plsc.load_gather(p2_b, [cv])
                    match = jnp.logical_and(
                        kv == tv, lax.shift_right_logical(val, 4) == p2v)
                    fidx = lax.bitwise_and(val, NBIN - 1) * C_PAD + cv
                    plsc.addupdate_scatter(cntb, [fidx], ones_i, mask=match)

                pltpu.sync_copy(cntb, hsh.at[pl.ds(sid * HROW, HROW)])
                plsc.subcore_barrier()
                pltpu.sync_copy(hsh, all_h)
                plsc.subcore_barrier()
                _reduce_hist(all_h, cntb)
                _radix_update(cntb, p2_b, r_b)

            @pl.loop(0, NVREG)
            def _(i):
                kv = keyb[pl.ds(i * 16, 16)]
                cv = catb[pl.ds(i * 16, 16)]
                tv = plsc.load_gather(p_b, [cv])
                tiv = plsc.load_gather(p2_b, [cv])
                asv = plsc.load_gather(as_b, [cv])
                jv = base + i * 16 + lax.broadcasted_iota(jnp.int32, (16,), 0)
                ltv = lax.bitwise_xor(kv, MININT) < lax.bitwise_xor(tv, MININT)
                selt = jnp.logical_and(kv == tv, jv < tiv)
                sel_v = jnp.logical_or(jnp.logical_or(ltv, selt), asv != 0)
                wb[pl.ds(i * 16, 16)] = jnp.where(
                    sel_v, jnp.float32(1.0), jnp.float32(0.0))

            pltpu.sync_copy(wb, w_hbm.at[pl.ds(base, CHUNK)])

            @pl.when(sid == 0)
            def _():
                pltpu.sync_copy(kf_b, kf_hbm)

        @pl.when(cid == 1)
        def _core1():
            per = BATCH // NSUB
            pltpu.sync_copy(u_hbm, ufull)
            pltpu.sync_copy(idx_hbm.at[pl.ds(sid * per, per)], ivb)

            @pl.loop(0, per // 16)
            def _(v):
                idxv = ivb[pl.ds(v * 16, 16)]
                ubb[pl.ds(v * 16, 16)] = plsc.load_gather(ufull, [idxv])

            pltpu.sync_copy(ubb, ub_hbm.at[pl.ds(sid * per, per)])

    return sel(u_pad, cat_pad, index, perc16)


SEG_BLK = 2000
SEG_GRID = NUM_EXAMP // SEG_BLK


def _segsum_body(cat_ref, w_ref, prev_ref, out_ref):
    i = pl.program_id(0)

    @pl.when(i == 0)
    def _():
        out_ref[...] = jnp.zeros_like(out_ref)

    c = cat_ref[0, 0, :]
    wv = w_ref[0, 0, :]
    cls = lax.broadcasted_iota(jnp.int32, (SEG_BLK, C_PAD), 1)
    onehot_w = jnp.where(c[:, None] == cls, wv[:, None], jnp.float32(0.0))
    out_ref[...] += lax.dot_general(
        onehot_w, prev_ref[...], (((0,), (0,)), ((), ())),
        preferred_element_type=jnp.float32,
        precision=lax.Precision.HIGHEST)


def _segsum(cat3, w3, prev):
    return pl.pallas_call(
        _segsum_body,
        grid=(SEG_GRID,),
        in_specs=[
            pl.BlockSpec((1, 1, SEG_BLK), lambda i: (i, 0, 0)),
            pl.BlockSpec((1, 1, SEG_BLK), lambda i: (i, 0, 0)),
            pl.BlockSpec((SEG_BLK, FEAT), lambda i: (i, 0)),
        ],
        out_specs=pl.BlockSpec((C_PAD, FEAT), lambda i: (0, 0)),
        out_shape=jax.ShapeDtypeStruct((C_PAD, FEAT), jnp.float32),
    )(cat3, w3, prev)


def _epilogue_body(outputs_ref, label_ref, out_ref, ub_ref, mvs_ref, kf_ref,
                   loss_ref):
    crow = lax.broadcasted_iota(jnp.int32, (C_PAD, 1), 0)
    cvalid = crow < NUM_CLASSES
    mv_sum = mvs_ref[...]
    kf = kf_ref[...]
    mv = jnp.where(cvalid, mv_sum / kf, jnp.float32(0.0))
    norm = jnp.sqrt(jnp.sum(mv * mv, axis=1, keepdims=True))
    norm = jnp.where(cvalid, norm, jnp.float32(1.0))
    mvn = mv / norm

    o = out_ref[...]
    onorm = o / jnp.sqrt(jnp.sum(o * o, axis=1, keepdims=True))
    sim = lax.dot_general(onorm, mvn, (((1,), (1,)), ((), ())),
                          preferred_element_type=jnp.float32,
                          precision=lax.Precision.HIGHEST)
    labelv = label_ref[...]
    sim = sim * labelv
    sim = sim * (sim > 0.0).astype(jnp.float32)

    logits = outputs_ref[...]
    rmax = jnp.max(logits, axis=1, keepdims=True)
    e = jnp.exp(logits - rmax)
    pred = e / jnp.sum(e, axis=1, keepdims=True)

    ub2 = ub_ref[...] * labelv
    predc = jnp.clip(pred + ub2, EPS, 1.0)
    loss = jnp.mean(-jnp.sum(sim * jnp.log(predc), axis=1))

    ccol = lax.broadcasted_iota(jnp.int32, (BATCH, C_PAD), 1)
    ismax = logits == rmax
    firsti = jnp.min(jnp.where(ismax, ccol, C_PAD), axis=1, keepdims=True)
    onehot = (ccol == firsti).astype(jnp.float32)
    mse = jnp.sum((onehot + ub2 - labelv) ** 2) / BATCH
    loss = loss + mse

    avgp = jnp.clip(jnp.mean(predc, axis=0, keepdims=True), EPS, 1.0)
    lg = jnp.where(ccol[0:1, :] < NUM_CLASSES, jnp.log(avgp), jnp.float32(0.0))
    balance_kl = -jnp.sum(lg) / NUM_CLASSES
    total = loss + jnp.float32(0.1) * balance_kl
    loss_ref[...] = jnp.reshape(total, (1, 1))


def _epilogue(outputs_pad, label_pad, out, ub, mv_sum, kf):
    return pl.pallas_call(
        _epilogue_body,
        out_shape=jax.ShapeDtypeStruct((1, 1), jnp.float32),
    )(outputs_pad, label_pad, out, ub, mv_sum, kf)


def kernel(index, outputs, label, out, u, prevSimilarity, masterVector,
           cat_labels, flag, epoch):
    del masterVector, flag
    percent = jnp.ceil(50 - 50.0 / 150.0 * epoch + 50).astype(jnp.float32)
    perc16 = jnp.full((16,), percent, jnp.float32)

    u_flat = u[:, 0]
    u_pad = jnp.concatenate([u_flat, jnp.zeros((N_PAD - NUM_EXAMP,), jnp.float32)])
    cat_pad = jnp.concatenate([
        cat_labels.astype(jnp.int32),
        jnp.full((N_PAD - NUM_EXAMP,), C_PAD - 1, jnp.int32)])

    w_pad, kf, ub = _sc_select(u_pad, cat_pad, index.astype(jnp.int32), perc16)

    cat3 = cat_labels.astype(jnp.int32).reshape(SEG_GRID, 1, SEG_BLK)
    w3 = w_pad[:NUM_EXAMP].reshape(SEG_GRID, 1, SEG_BLK)
    mv_sum = _segsum(cat3, w3, prevSimilarity)

    neg = jnp.full((BATCH, C_PAD - NUM_CLASSES), -jnp.inf, jnp.float32)
    outputs_pad = jnp.concatenate([outputs, neg], axis=1)
    label_pad = jnp.concatenate(
        [label, jnp.zeros((BATCH, C_PAD - NUM_CLASSES), jnp.float32)], axis=1)

    loss = _epilogue(outputs_pad, label_pad, out, ub.reshape(BATCH, 1),
                     mv_sum, kf.reshape(C_PAD, 1))
    return loss[0, 0]


# speedup vs baseline: 61.0057x; 1.1850x over previous
import dataclasses
import functools

import jax
import jax.numpy as jnp
import numpy as np
from jax import lax
from jax.experimental import pallas as pl
from jax.experimental.pallas import tpu as pltpu
from jax.experimental.pallas import tpu_sc as plsc

NUM_EXAMP = 50000
NUM_CLASSES = 100
BATCH = 1024
FEAT = 512
EPS = 1e-4

NSUB = 16
N_PAD = 50176
CHUNK = N_PAD // NSUB
NVREG = CHUNK // 16
C_PAD = 112
NBIN = 16
HROW = NBIN * C_PAD
MININT = np.int32(-2147483648)


def _radix_scan(hrd2d, p_ref, r_ref):
    for cg in range(C_PAD // 16):
        r_v = r_ref[pl.ds(cg * 16, 16)]
        p_v = p_ref[pl.ds(cg * 16, 16)]
        cum = r_v ^ r_v
        bsel = cum
        newr = r_v
        done = cum == ones16(cum)
        for b in range(NBIN):
            tot = hrd2d[7 * b + cg, pl.ds(0, 16)]
            prev_cum = cum
            cum = cum + tot
            take = jnp.logical_and(jnp.logical_not(done), r_v < cum)
            bsel = jnp.where(take, jnp.int32(b), bsel)
            newr = jnp.where(take, r_v - prev_cum, newr)
            done = jnp.logical_or(done, take)
        p_ref[pl.ds(cg * 16, 16)] = p_v * NBIN + bsel
        r_ref[pl.ds(cg * 16, 16)] = newr


def ones16(like):
    return (like ^ like) + 1


def _sc_select(u_pad, cat_pad, index, perc16):
    mesh = plsc.VectorSubcoreMesh(core_axis_name="c", subcore_axis_name="s")
    cp = pltpu.CompilerParams()
    if "needs_layout_passes" in pltpu.CompilerParams.__dataclass_fields__:
        cp = dataclasses.replace(cp, needs_layout_passes=False)

    @functools.partial(
        pl.kernel,
        mesh=mesh,
        compiler_params=cp,
        out_type=[
            jax.ShapeDtypeStruct((N_PAD,), jnp.float32),
            jax.ShapeDtypeStruct((C_PAD,), jnp.float32),
            jax.ShapeDtypeStruct((BATCH,), jnp.float32),
        ],
        scratch_types=[
            pltpu.VMEM((CHUNK,), jnp.float32),
            pltpu.VMEM((CHUNK,), jnp.int32),
            pltpu.VMEM((CHUNK,), jnp.int32),
            pltpu.VMEM((CHUNK,), jnp.float32),
            pltpu.VMEM((C_PAD,), jnp.int32),
            pltpu.VMEM((C_PAD,), jnp.int32),
            pltpu.VMEM((C_PAD,), jnp.int32),
            pltpu.VMEM((C_PAD,), jnp.int32),
            pltpu.VMEM((C_PAD,), jnp.float32),
            pltpu.VMEM((HROW // 16, 16), jnp.int32),
            pltpu.VMEM((HROW // 16, 16), jnp.int32),
            pltpu.VMEM((HROW // 16,), jnp.int32),
            pltpu.VMEM((7, 16), jnp.int32),
            pltpu.VMEM((16,), jnp.float32),
            pltpu.VMEM((N_PAD,), jnp.float32),
            pltpu.VMEM((BATCH // NSUB,), jnp.int32),
            pltpu.VMEM((BATCH // NSUB,), jnp.float32),
            pltpu.VMEM_SHARED((HROW // 16, 16), jnp.int32),
            pltpu.VMEM_SHARED((HROW // 16, 16), jnp.int32),
        ],
    )
    def sel(u_hbm, cat_hbm, idx_hbm, perc_hbm, w_hbm, kf_hbm, ub_hbm,
            utmp, keyb, catb, wb, p_b, p2_b, r_b, as_b, kf_b,
            cnt2d, hrd2d, idxrows, zrows, perc, ufull, ivb, ubb, hsha, hshb):
        cid = lax.axis_index("c")
        sid = lax.axis_index("s")
        ones_i = jnp.ones((16,), jnp.int32)
        zeros_i = jnp.zeros((16,), jnp.int32)
        iota16 = lax.broadcasted_iota(jnp.int32, (16,), 0)

        @pl.when(cid == 0)
        def _core0():
            base = sid * CHUNK
            pltpu.sync_copy(u_hbm.at[pl.ds(base, CHUNK)], utmp)
            pltpu.sync_copy(cat_hbm.at[pl.ds(base, CHUNK)], catb)
            pltpu.sync_copy(perc_hbm, perc)

            @pl.loop(0, 7)
            def _(q):
                idxrows[pl.ds(q * 16, 16)] = iota16 + q * 16
                zrows[q, pl.ds(0, 16)] = zeros_i

            @pl.loop(0, NVREG)
            def _(i):
                fb = plsc.bitcast(utmp[pl.ds(i * 16, 16)], jnp.int32)
                m = lax.shift_right_arithmetic(fb, 31)
                keyb[pl.ds(i * 16, 16)] = lax.bitwise_xor(
                    fb, lax.bitwise_or(m, MININT))

            def zero_cnt():
                @pl.loop(0, HROW // 16)
                def _(q):
                    cnt2d[q, pl.ds(0, 16)] = zeros_i

            def combine(g):
                buf = hsha if g % 2 == 0 else hshb
                plsc.subcore_barrier()
                pltpu.sync_copy(cnt2d, buf.at[idxrows], add=True)
                plsc.subcore_barrier()
                pltpu.sync_copy(buf, hrd2d)

            def prezero(g):
                buf = hsha if g % 2 == 0 else hshb
                pltpu.sync_copy(zrows, buf.at[pl.ds(sid * 7, 7)])

            for rnd in range(8):
                shift = 32 - 4 * (rnd + 1)
                prezero(rnd)
                zero_cnt()

                @pl.loop(0, NVREG)
                def _(i, _shift=shift):
                    kv = keyb[pl.ds(i * 16, 16)]
                    cv = catb[pl.ds(i * 16, 16)]
                    val = lax.shift_right_logical(kv, _shift)
                    binv = lax.bitwise_and(val, NBIN - 1)
                    rowv = binv * 7 + lax.shift_right_logical(cv, 4)
                    colv = lax.bitwise_and(cv, 15)
                    if _shift == 28:
                        plsc.addupdate_scatter(cnt2d, [rowv, colv], ones_i)
                    else:
                        pv = plsc.load_gather(p_b, [cv])
                        match = lax.shift_right_logical(val, 4) == pv
                        plsc.addupdate_scatter(cnt2d, [rowv, colv], ones_i,
                                               mask=match)

                combine(rnd)
                if rnd == 0:
                    pv16 = perc[pl.ds(0, 16)]
                    for cg in range(C_PAD // 16):
                        n_v = hrd2d[cg, pl.ds(0, 16)]
                        for b in range(1, NBIN):
                            n_v = n_v + hrd2d[7 * b + cg, pl.ds(0, 16)]
                        nf = n_v.astype(jnp.float32)
                        ki = ((nf / jnp.float32(100.0)) * pv16).astype(jnp.int32)
                        kf_b[pl.ds(cg * 16, 16)] = ki.astype(jnp.float32)
                        r_b[pl.ds(cg * 16, 16)] = ki
                        p_b[pl.ds(cg * 16, 16)] = zeros_i
                        as_b[pl.ds(cg * 16, 16)] = (ki >= n_v).astype(jnp.int32)
                _radix_scan(hrd2d, p_b, r_b)

            for cg in range(C_PAD // 16):
                p2_b[pl.ds(cg * 16, 16)] = zeros_i
            for rnd in range(4):
                shift = 16 - 4 * (rnd + 1)
                g = 8 + rnd
                prezero(g)
                zero_cnt()

                @pl.loop(0, NVREG)
                def _(i, _shift=shift):
                    kv = keyb[pl.ds(i * 16, 16)]
                    cv = catb[pl.ds(i * 16, 16)]
                    tv = plsc.load_gather(p_b, [cv])
                    jv = base + i * 16 + iota16
                    val = lax.shift_right_logical(jv, _shift)
                    p2v = plsc.load_gather(p2_b, [cv])
                    match = jnp.logical_and(
                        kv == tv, lax.shift_right_logical(val, 4) == p2v)
                    binv = lax.bitwise_and(val, NBIN - 1)
                    rowv = binv * 7 + lax.shift_right_logical(cv, 4)
                    colv = lax.bitwise_and(cv, 15)
                    plsc.addupdate_scatter(cnt2d, [rowv, colv], ones_i,
                                           mask=match)

                combine(g)
                _radix_scan(hrd2d, p2_b, r_b)

            @pl.loop(0, NVREG)
            def _(i):
                kv = keyb[pl.ds(i * 16, 16)]
                cv = catb[pl.ds(i * 16, 16)]
                tv = plsc.load_gather(p_b, [cv])
                tiv = plsc.load_gather(p2_b, [cv])
                asv = plsc.load_gather(as_b, [cv])
                jv = base + i * 16 + iota16
                ltv = lax.bitwise_xor(kv, MININT) < lax.bitwise_xor(tv, MININT)
                selt = jnp.logical_and(kv == tv, jv < tiv)
                sel_v = jnp.logical_or(jnp.logical_or(ltv, selt), asv != 0)
                wb[pl.ds(i * 16, 16)] = jnp.where(
                    sel_v, jnp.float32(1.0), jnp.float32(0.0))

            pltpu.sync_copy(wb, w_hbm.at[pl.ds(base, CHUNK)])

            @pl.when(sid == 0)
            def _():
                pltpu.sync_copy(kf_b, kf_hbm)

        @pl.when(cid == 1)
        def _core1():
            per = BATCH // NSUB
            pltpu.sync_copy(u_hbm, ufull)
            pltpu.sync_copy(idx_hbm.at[pl.ds(sid * per, per)], ivb)

            @pl.loop(0, per // 16)
            def _(v):
                idxv = ivb[pl.ds(v * 16, 16)]
                ubb[pl.ds(v * 16, 16)] = plsc.load_gather(ufull, [idxv])

            pltpu.sync_copy(ubb, ub_hbm.at[pl.ds(sid * per, per)])

    return sel(u_pad, cat_pad, index, perc16)


SEG_BLK = 2000
SEG_GRID = NUM_EXAMP // SEG_BLK


def _segsum_body(cat_ref, w_ref, prev_ref, out_ref):
    i = pl.program_id(0)

    @pl.when(i == 0)
    def _():
        out_ref[...] = jnp.zeros_like(out_ref)

    c = cat_ref[0, 0, :]
    wv = w_ref[0, 0, :]
    cls = lax.broadcasted_iota(jnp.int32, (SEG_BLK, C_PAD), 1)
    onehot_w = jnp.where(c[:, None] == cls, wv[:, None], jnp.float32(0.0))
    out_ref[...] += lax.dot_general(
        onehot_w, prev_ref[...], (((0,), (0,)), ((), ())),
        preferred_element_type=jnp.float32,
        precision=lax.Precision.HIGHEST)


def _segsum(cat3, w3, prev):
    return pl.pallas_call(
        _segsum_body,
        grid=(SEG_GRID,),
        in_specs=[
            pl.BlockSpec((1, 1, SEG_BLK), lambda i: (i, 0, 0)),
            pl.BlockSpec((1, 1, SEG_BLK), lambda i: (i, 0, 0)),
            pl.BlockSpec((SEG_BLK, FEAT), lambda i: (i, 0)),
        ],
        out_specs=pl.BlockSpec((C_PAD, FEAT), lambda i: (0, 0)),
        out_shape=jax.ShapeDtypeStruct((C_PAD, FEAT), jnp.float32),
    )(cat3, w3, prev)


def _epilogue_body(outputs_ref, label_ref, out_ref, ub_ref, mvs_ref, kf_ref,
                   loss_ref):
    crow = lax.broadcasted_iota(jnp.int32, (C_PAD, 1), 0)
    cvalid = crow < NUM_CLASSES
    mv_sum = mvs_ref[...]
    kf = kf_ref[...]
    mv = jnp.where(cvalid, mv_sum / kf, jnp.float32(0.0))
    norm = jnp.sqrt(jnp.sum(mv * mv, axis=1, keepdims=True))
    norm = jnp.where(cvalid, norm, jnp.float32(1.0))
    mvn = mv / norm

    o = out_ref[...]
    onorm = o / jnp.sqrt(jnp.sum(o * o, axis=1, keepdims=True))
    sim = lax.dot_general(onorm, mvn, (((1,), (1,)), ((), ())),
                          preferred_element_type=jnp.float32,
                          precision=lax.Precision.HIGHEST)
    labelv = label_ref[...]
    sim = sim * labelv
    sim = sim * (sim > 0.0).astype(jnp.float32)

    logits = outputs_ref[...]
    rmax = jnp.max(logits, axis=1, keepdims=True)
    e = jnp.exp(logits - rmax)
    pred = e / jnp.sum(e, axis=1, keepdims=True)

    ub2 = ub_ref[...] * labelv
    predc = jnp.clip(pred + ub2, EPS, 1.0)
    loss = jnp.mean(-jnp.sum(sim * jnp.log(predc), axis=1))

    ccol = lax.broadcasted_iota(jnp.int32, (BATCH, C_PAD), 1)
    ismax = logits == rmax
    firsti = jnp.min(jnp.where(ismax, ccol, C_PAD), axis=1, keepdims=True)
    onehot = (ccol == firsti).astype(jnp.float32)
    mse = jnp.sum((onehot + ub2 - labelv) ** 2) / BATCH
    loss = loss + mse

    avgp = jnp.clip(jnp.mean(predc, axis=0, keepdims=True), EPS, 1.0)
    lg = jnp.where(ccol[0:1, :] < NUM_CLASSES, jnp.log(avgp), jnp.float32(0.0))
    balance_kl = -jnp.sum(lg) / NUM_CLASSES
    total = loss + jnp.float32(0.1) * balance_kl
    loss_ref[...] = jnp.reshape(total, (1, 1))


def _epilogue(outputs_pad, label_pad, out, ub, mv_sum, kf):
    return pl.pallas_call(
        _epilogue_body,
        out_shape=jax.ShapeDtypeStruct((1, 1), jnp.float32),
    )(outputs_pad, label_pad, out, ub, mv_sum, kf)


def kernel(index, outputs, label, out, u, prevSimilarity, masterVector,
           cat_labels, flag, epoch):
    del masterVector, flag
    percent = jnp.ceil(50 - 50.0 / 150.0 * epoch + 50).astype(jnp.float32)
    perc16 = jnp.full((16,), percent, jnp.float32)

    u_flat = u[:, 0]
    u_pad = jnp.concatenate([u_flat, jnp.zeros((N_PAD - NUM_EXAMP,), jnp.float32)])
    cat_pad = jnp.concatenate([
        cat_labels.astype(jnp.int32),
        jnp.full((N_PAD - NUM_EXAMP,), C_PAD - 1, jnp.int32)])

    w_pad, kf, ub = _sc_select(u_pad, cat_pad, index.astype(jnp.int32), perc16)

    cat3 = cat_labels.astype(jnp.int32).reshape(SEG_GRID, 1, SEG_BLK)
    w3 = w_pad[:NUM_EXAMP].reshape(SEG_GRID, 1, SEG_BLK)
    mv_sum = _segsum(cat3, w3, prevSimilarity)

    neg = jnp.full((BATCH, C_PAD - NUM_CLASSES), -jnp.inf, jnp.float32)
    outputs_pad = jnp.concatenate([outputs, neg], axis=1)
    label_pad = jnp.concatenate(
        [label, jnp.zeros((BATCH, C_PAD - NUM_CLASSES), jnp.float32)], axis=1)

    loss = _epilogue(outputs_pad, label_pad, out, ub.reshape(BATCH, 1),
                     mv_sum, kf.reshape(C_PAD, 1))
    return loss[0, 0]
